# Initial kernel scaffold; baseline (speedup 1.0000x reference)
import jax
import jax.numpy as jnp
from jax.experimental import pallas as pl


def kernel(feat, edge_index, W, b):
    raise NotImplementedError("write your pallas kernel here")


# speedup vs baseline: 10.2605x; 10.2605x over previous
import functools

import jax
import jax.numpy as jnp
from jax import lax
from jax.experimental import pallas as pl
from jax.experimental.pallas import tpu as pltpu
from jax.experimental.pallas import tpu_sc as plsc

N_NODES = 10000
D = 128
HD = D // 2
E = 320000
NC = 2
NS = 16
CHUNK = 128
K = 160
EPW = K * CHUNK
E_PAD = NS * EPW
ACC_PAD = 16
ACC_ROWS = N_NODES + ACC_PAD
SPAN = 624

_mesh = plsc.VectorSubcoreMesh(core_axis_name="c", subcore_axis_name="s")


@functools.partial(
    pl.kernel,
    mesh=_mesh,
    out_type=jax.ShapeDtypeStruct((NC, N_NODES, HD), jnp.float32),
    scratch_types=[
        pltpu.VMEM((K, CHUNK), jnp.int32),
        pltpu.VMEM((K, CHUNK), jnp.int32),
        pltpu.VMEM((CHUNK, HD), jnp.float32),
        pltpu.VMEM((CHUNK, HD), jnp.float32),
        pltpu.VMEM_SHARED((ACC_ROWS, HD), jnp.float32),
        pltpu.SemaphoreType.DMA,
        pltpu.SemaphoreType.DMA,
    ],
    compiler_params=pltpu.CompilerParams(use_tc_tiling_on_sc=False),
)
def _sc_aggregate(src_hbm, dst_hbm, feat_hbm, out_hbm,
                  src_v, dst_v, buf_a, buf_b, acc, sem_a, sem_b):
    c = lax.axis_index("c")
    s = lax.axis_index("s")

    zero16 = jnp.zeros((16,), jnp.float32)

    def _zbody(i, carry):
        buf_a[i // (HD // 16), pl.ds((i % (HD // 16)) * 16, 16)] = zero16
        return carry

    lax.fori_loop(0, (CHUNK * HD) // 16, _zbody, 0)

    zbase = s * SPAN
    pltpu.sync_copy(buf_a, acc.at[pl.ds(zbase, CHUNK)])
    pltpu.sync_copy(buf_a, acc.at[pl.ds(zbase + CHUNK, CHUNK)])
    pltpu.sync_copy(buf_a, acc.at[pl.ds(zbase + 2 * CHUNK, CHUNK)])
    pltpu.sync_copy(buf_a, acc.at[pl.ds(zbase + 3 * CHUNK, CHUNK)])
    pltpu.sync_copy(buf_a.at[pl.ds(0, SPAN - 4 * CHUNK)],
                    acc.at[pl.ds(zbase + 4 * CHUNK, SPAN - 4 * CHUNK)])

    @pl.when(s == NS - 1)
    def _ztail():
        pltpu.sync_copy(buf_a.at[pl.ds(0, ACC_ROWS - NS * SPAN)],
                        acc.at[pl.ds(NS * SPAN, ACC_ROWS - NS * SPAN)])

    plsc.subcore_barrier()

    pltpu.sync_copy(src_hbm.at[s], src_v)
    pltpu.sync_copy(dst_hbm.at[s], dst_v)

    feat_c = feat_hbm.at[c]
    pltpu.async_copy(feat_c.at[src_v.at[0]], buf_a, sem_a)

    def _body(jj, carry):
        j0 = 2 * jj
        pltpu.async_copy(feat_c.at[src_v.at[j0 + 1]], buf_b, sem_b)
        pltpu.make_async_copy(feat_c.at[src_v.at[j0]], buf_a, sem_a).wait()
        pltpu.sync_copy(buf_a, acc.at[dst_v.at[j0]], add=True)

        @pl.when(jj < K // 2 - 1)
        def _():
            pltpu.async_copy(feat_c.at[src_v.at[j0 + 2]], buf_a, sem_a)

        pltpu.make_async_copy(feat_c.at[src_v.at[j0 + 1]], buf_b, sem_b).wait()
        pltpu.sync_copy(buf_b, acc.at[dst_v.at[j0 + 1]], add=True)
        return carry

    lax.fori_loop(0, K // 2, _body, 0)

    plsc.subcore_barrier()

    obase = s * SPAN
    pltpu.sync_copy(acc.at[pl.ds(obase, SPAN)],
                    out_hbm.at[c].at[pl.ds(obase, SPAN)])

    @pl.when(s == NS - 1)
    def _otail():
        pltpu.sync_copy(acc.at[pl.ds(NS * SPAN, N_NODES - NS * SPAN)],
                        out_hbm.at[c].at[pl.ds(NS * SPAN, N_NODES - NS * SPAN)])


BLK = 1000


def _tc_body(p_ref, wt_ref, b_ref, o_ref):
    agg = jnp.concatenate([p_ref[0], p_ref[1]], axis=-1)
    o_ref[...] = (
        jnp.dot(agg, wt_ref[...], preferred_element_type=jnp.float32)
        + b_ref[...]
    )


def _tc_linear(partials, wt, b2):
    return pl.pallas_call(
        _tc_body,
        grid=(N_NODES // BLK,),
        in_specs=[
            pl.BlockSpec((NC, BLK, HD), lambda i: (0, i, 0)),
            pl.BlockSpec((D, D), lambda i: (0, 0)),
            pl.BlockSpec((1, D), lambda i: (0, 0)),
        ],
        out_specs=pl.BlockSpec((BLK, D), lambda i: (i, 0)),
        out_shape=jax.ShapeDtypeStruct((N_NODES, D), jnp.float32),
    )(partials, wt, b2)


def kernel(feat, edge_index, W, b):
    src = edge_index[0].astype(jnp.int32)
    dst = edge_index[1].astype(jnp.int32)
    pad = E_PAD - E
    ar = jnp.arange(pad, dtype=jnp.int32)
    src_p = jnp.concatenate([src, ar % jnp.int32(N_NODES)])
    dst_p = jnp.concatenate([dst, jnp.int32(N_NODES) + (ar % jnp.int32(ACC_PAD))])
    src3 = src_p.reshape(NS, K, CHUNK)
    dst3 = dst_p.reshape(NS, K, CHUNK)
    feat_halves = jnp.stack([feat[:, :HD], feat[:, HD:]])
    partials = _sc_aggregate(src3, dst3, feat_halves)
    return _tc_linear(partials, W.T, b.reshape(1, D))


# speedup vs baseline: 11.5510x; 1.1258x over previous
import functools

import jax
import jax.numpy as jnp
from jax import lax
from jax.experimental import pallas as pl
from jax.experimental.pallas import tpu as pltpu
from jax.experimental.pallas import tpu_sc as plsc

N_NODES = 10000
D = 128
HD = D // 2
E = 320000
NC = 2
NS = 16
CHUNK = 128
K = 160
GE = 256
NG = (K * CHUNK) // GE
EPW = K * CHUNK
E_PAD = NS * EPW
ACC_PAD = 16
ACC_ROWS = N_NODES + ACC_PAD
SPAN = 624

_mesh = plsc.VectorSubcoreMesh(core_axis_name="c", subcore_axis_name="s")


@functools.partial(
    pl.kernel,
    mesh=_mesh,
    out_type=jax.ShapeDtypeStruct((NC, N_NODES, HD), jnp.float32),
    scratch_types=[
        pltpu.VMEM((EPW,), jnp.int32),
        pltpu.VMEM((EPW,), jnp.int32),
        pltpu.VMEM((GE, HD), jnp.float32),
        pltpu.VMEM((GE, HD), jnp.float32),
        pltpu.VMEM_SHARED((ACC_ROWS, HD), jnp.float32),
        pltpu.SemaphoreType.DMA,
        pltpu.SemaphoreType.DMA,
    ],
    compiler_params=pltpu.CompilerParams(use_tc_tiling_on_sc=False),
)
def _sc_aggregate(src_hbm, dst_hbm, feat_hbm, out_hbm,
                  src_v, dst_v, buf_a, buf_b, acc, sem_a, sem_b):
    c = lax.axis_index("c")
    s = lax.axis_index("s")

    zero16 = jnp.zeros((16,), jnp.float32)

    def _zbody(i, carry):
        buf_a[i // (HD // 16), pl.ds((i % (HD // 16)) * 16, 16)] = zero16
        return carry

    lax.fori_loop(0, (CHUNK * HD) // 16, _zbody, 0)

    zpage = buf_a.at[pl.ds(0, CHUNK)]
    zbase = s * SPAN
    pltpu.sync_copy(zpage, acc.at[pl.ds(zbase, CHUNK)])
    pltpu.sync_copy(zpage, acc.at[pl.ds(zbase + CHUNK, CHUNK)])
    pltpu.sync_copy(zpage, acc.at[pl.ds(zbase + 2 * CHUNK, CHUNK)])
    pltpu.sync_copy(zpage, acc.at[pl.ds(zbase + 3 * CHUNK, CHUNK)])
    pltpu.sync_copy(zpage.at[pl.ds(0, SPAN - 4 * CHUNK)],
                    acc.at[pl.ds(zbase + 4 * CHUNK, SPAN - 4 * CHUNK)])

    @pl.when(s == NS - 1)
    def _ztail():
        pltpu.sync_copy(zpage.at[pl.ds(0, ACC_ROWS - NS * SPAN)],
                        acc.at[pl.ds(NS * SPAN, ACC_ROWS - NS * SPAN)])

    plsc.subcore_barrier()

    pltpu.sync_copy(src_hbm.at[s], src_v)
    pltpu.sync_copy(dst_hbm.at[s], dst_v)

    feat_c = feat_hbm.at[c]

    def _sidx(g):
        return src_v.at[pl.ds(g * GE, GE)]

    def _didx(g):
        return dst_v.at[pl.ds(g * GE, GE)]

    pltpu.async_copy(feat_c.at[_sidx(0)], buf_a, sem_a)

    def _body(jj, carry):
        g0 = 2 * jj
        pltpu.async_copy(feat_c.at[_sidx(g0 + 1)], buf_b, sem_b)
        pltpu.make_async_copy(feat_c.at[_sidx(g0)], buf_a, sem_a).wait()
        pltpu.sync_copy(buf_a, acc.at[_didx(g0)], add=True)

        @pl.when(jj < NG // 2 - 1)
        def _():
            pltpu.async_copy(feat_c.at[_sidx(g0 + 2)], buf_a, sem_a)

        pltpu.make_async_copy(feat_c.at[_sidx(g0 + 1)], buf_b, sem_b).wait()
        pltpu.sync_copy(buf_b, acc.at[_didx(g0 + 1)], add=True)
        return carry

    lax.fori_loop(0, NG // 2, _body, 0)

    plsc.subcore_barrier()

    obase = s * SPAN
    pltpu.sync_copy(acc.at[pl.ds(obase, SPAN)],
                    out_hbm.at[c].at[pl.ds(obase, SPAN)])

    @pl.when(s == NS - 1)
    def _otail():
        pltpu.sync_copy(acc.at[pl.ds(NS * SPAN, N_NODES - NS * SPAN)],
                        out_hbm.at[c].at[pl.ds(NS * SPAN, N_NODES - NS * SPAN)])


BLK = 1000


def _tc_body(p_ref, wt_ref, b_ref, o_ref):
    agg = jnp.concatenate([p_ref[0], p_ref[1]], axis=-1)
    o_ref[...] = (
        jnp.dot(agg, wt_ref[...], preferred_element_type=jnp.float32)
        + b_ref[...]
    )


def _tc_linear(partials, wt, b2):
    return pl.pallas_call(
        _tc_body,
        grid=(N_NODES // BLK,),
        in_specs=[
            pl.BlockSpec((NC, BLK, HD), lambda i: (0, i, 0)),
            pl.BlockSpec((D, D), lambda i: (0, 0)),
            pl.BlockSpec((1, D), lambda i: (0, 0)),
        ],
        out_specs=pl.BlockSpec((BLK, D), lambda i: (i, 0)),
        out_shape=jax.ShapeDtypeStruct((N_NODES, D), jnp.float32),
    )(partials, wt, b2)


def kernel(feat, edge_index, W, b):
    src = edge_index[0].astype(jnp.int32)
    dst = edge_index[1].astype(jnp.int32)
    pad = E_PAD - E
    ar = jnp.arange(pad, dtype=jnp.int32)
    src_p = jnp.concatenate([src, ar % jnp.int32(N_NODES)])
    dst_p = jnp.concatenate([dst, jnp.int32(N_NODES) + (ar % jnp.int32(ACC_PAD))])
    src3 = src_p.reshape(NS, EPW)
    dst3 = dst_p.reshape(NS, EPW)
    feat_halves = jnp.stack([feat[:, :HD], feat[:, HD:]])
    partials = _sc_aggregate(src3, dst3, feat_halves)
    return _tc_linear(partials, W.T, b.reshape(1, D))
